# baseline (speedup 1.0000x reference)
import functools

import jax
import jax.numpy as jnp
from jax import lax
from jax.experimental import pallas as pl
from jax.experimental.pallas import tpu as pltpu
from jax.experimental.pallas import tpu_sc as plsc

V = 1000000
D = 64
B = 16384
L = 16
NC, NS = 2, 16
NW = NC * NS

CHW = 512
NCH = 61
SPAN = NCH * CHW
LEFT_LO = 999424
XT_LO = 999936

WCAP = 832
WPAD = 912
RING = 128

_MAGIC = 0x5F3759DF
_EPS = 1e-5


@functools.partial(
    pl.kernel,
    mesh=plsc.VectorSubcoreMesh(core_axis_name="c", subcore_axis_name="s"),
    out_type=jax.ShapeDtypeStruct((B * D,), jnp.float32),
    scratch_types=[
        pltpu.VMEM((B,), jnp.int32),
        pltpu.VMEM((WPAD,), jnp.int32),
        pltpu.VMEM((WPAD,), jnp.int32),
        pltpu.VMEM((D, CHW), jnp.float32),
        pltpu.VMEM((D, CHW), jnp.float32),
        pltpu.VMEM((RING * D,), jnp.float32),
        pltpu.VMEM((2 * D,), jnp.float32),
        pltpu.VMEM((2 * L,), jnp.float32),
        pltpu.SemaphoreType.DMA,
        pltpu.SemaphoreType.DMA,
        pltpu.SemaphoreType.DMA,
    ],
)
def _embed_ln_kernel(tok_hbm, tt_hbm, gb_hbm, sc_hbm, tl_hbm, out_hbm,
                     ids_v, wid_v, wpos_v, buf0, buf1,
                     rows_v, gb_v, sc_v, gsem0, gsem1, osem):
    w = lax.axis_index("s") * NC + lax.axis_index("c")
    lo = w * SPAN

    pltpu.sync_copy(tok_hbm, ids_v)
    pltpu.sync_copy(gb_hbm, gb_v)
    pltpu.sync_copy(sc_hbm, sc_v)

    lane = lax.iota(jnp.int32, L)

    dnums = lax.GatherDimensionNumbers(
        offset_dims=(), collapsed_slice_dims=(0,), start_index_map=(0,))

    def permute(x, p):
        return lax.gather(x, p[:, None], dnums, slice_sizes=(1,),
                          mode=lax.GatherScatterMode.PROMISE_IN_BOUNDS)

    xorperms = [lane ^ m for m in (8, 4, 2, 1)]

    def hsum(x):
        for p in xorperms:
            x = x + permute(x, p)
        return x

    shperms = [jnp.maximum(lane - sh, 0) for sh in (1, 2, 4, 8)]
    shsels = [jnp.where(lane >= sh, 1, 0) for sh in (1, 2, 4, 8)]

    def incl_prefix(x):
        for p, s in zip(shperms, shsels):
            x = x + jnp.where(s == 1, permute(x, p), 0)
        return x

    def rsqrt_vec(v):
        i = lax.bitcast_convert_type(v, jnp.int32)
        y = lax.bitcast_convert_type(jnp.int32(_MAGIC) - (i >> 1),
                                     jnp.float32)
        hv = v * 0.5
        for _ in range(3):
            y = y * (1.5 - hv * y * y)
        return y

    is_last = w == NW - 1
    elo = jnp.where(is_last, LEFT_LO, V)

    def scan1(b, cur):
        ids = ids_v[pl.ds(b * L, L)]
        poss = lane + b * L
        m = ((ids >= lo) & (ids < lo + SPAN)) | (ids >= elo)
        m01 = jnp.where(m, 1, 0)
        tot = hsum(m01)[0]

        def take(i, cur2):
            pref = incl_prefix(m01)
            sel = jnp.where((m01 == 1) & (pref == i + 1), 1, 0)
            tid = hsum(sel * ids)[0]
            tpos = hsum(sel * poss)[0]
            dst = jnp.minimum(cur2, WCAP - 1)
            wid_v[pl.ds(dst, L)] = jnp.full((L,), tid, jnp.int32)
            wpos_v[pl.ds(dst, L)] = jnp.full((L,), tpos, jnp.int32)
            return cur2 + 1

        return lax.fori_loop(0, tot, take, cur)

    def scan(b2, cur):
        for u in range(8):
            cur = scan1(b2 * 8 + u, cur)
        return cur

    n = lax.fori_loop(0, B // L // 8, scan, 0)
    n = jnp.minimum(n, WCAP)
    nv = (n + L - 1) // L

    gvecs = [gb_v[pl.ds(L * k, L)] for k in range(4)]
    bvecs = [gb_v[pl.ds(D + L * k, L)] for k in range(4)]
    wsv = sc_v[pl.ds(0, L)]
    nsv = sc_v[pl.ds(L, L)]
    lmasks = [jnp.where(lane == j, 1, 0) for j in range(L)]

    def ln_emit(x, tpos, cnt):
        s = (x[0] + x[1]) + (x[2] + x[3])
        q = (x[0] * x[0] + x[1] * x[1]) + (x[2] * x[2] + x[3] * x[3])
        mv = hsum(s) * (1.0 / D)
        ex2 = hsum(q) * (1.0 / D)
        var = ex2 - mv * mv
        rinv = rsqrt_vec(var + _EPS)

        @pl.when(cnt >= RING)
        def _():
            pltpu.make_async_copy(rows_v.at[pl.ds(0, D)],
                                  out_hbm.at[pl.ds(0, D)], osem).wait()

        slot = cnt & (RING - 1)
        for k in range(4):
            xh = (x[k] - mv) * rinv
            rows_v[pl.ds(slot * D + L * k, L)] = (
                x[k] * wsv + (xh * gvecs[k] + bvecs[k]) * nsv)
        pltpu.async_copy(rows_v.at[pl.ds(slot * D, D)],
                         out_hbm.at[pl.ds(tpos * D, D)], osem)

    def emit_token(buf, col, tpos, cnt):
        cw = (col >> 4) * L
        cl = jnp.full((L,), col & (L - 1), jnp.int32)
        x = []
        for k in range(4):
            acc = jnp.zeros((L,), jnp.float32)
            for j in range(L):
                spl = permute(buf[L * k + j, pl.ds(cw, L)], cl)
                acc = jnp.where(lmasks[j] == 1, spl, acc)
            x.append(acc)
        ln_emit(x, tpos, cnt)

    def chunk_tokens(buf, clo, chi, cnt):
        def vreg(vi, cnt2):
            ids = wid_v[pl.ds(vi * L, L)]
            inw = (lane + vi * L) < n
            m01 = jnp.where((ids >= clo) & (ids < chi) & inw, 1, 0)
            tot = hsum(m01)[0]

            def take(i, cnt3):
                pref = incl_prefix(m01)
                poss = wpos_v[pl.ds(vi * L, L)]
                sel = jnp.where((m01 == 1) & (pref == i + 1), 1, 0)
                tid = hsum(sel * ids)[0]
                tpos = hsum(sel * poss)[0]
                emit_token(buf, tid - clo, tpos, cnt3)
                return cnt3 + 1

            return lax.fori_loop(0, tot, take, cnt2)

        def vreg4(v4, cnt2):
            for u in range(4):
                cnt2 = vreg(4 * v4 + u, cnt2)
            return cnt2

        return lax.fori_loop(0, (nv + 3) // 4, vreg4, cnt)

    nch_w = NCH + jnp.where(is_last, 2, 0)

    def chunk_lo(c):
        return jnp.where(c < NCH, lo + c * CHW, LEFT_LO + (c - NCH) * CHW)

    def start_chunk(c, buf, sem):
        @pl.when(c < NCH)
        def _main():
            off = pl.multiple_of(lo + c * CHW, 128)
            pltpu.async_copy(tt_hbm.at[:, pl.ds(off, CHW)], buf, sem)

        @pl.when((c >= NCH) & (c < nch_w))
        def _left():
            off = pl.multiple_of((c - NCH) * CHW, 128)
            pltpu.async_copy(tl_hbm.at[:, pl.ds(off, CHW)], buf, sem)

    def wait_chunk(c, buf, sem):
        @pl.when(c < nch_w)
        def _():
            pltpu.make_async_copy(tt_hbm.at[:, pl.ds(0, CHW)],
                                  buf, sem).wait()

    start_chunk(0, buf0, gsem0)
    start_chunk(1, buf1, gsem1)

    def chunk_iter(k, cnt):
        c0 = 2 * k
        clo0 = chunk_lo(c0)
        wait_chunk(c0, buf0, gsem0)
        cnt = chunk_tokens(buf0, clo0, clo0 + CHW, cnt)
        start_chunk(c0 + 2, buf0, gsem0)

        c1 = c0 + 1
        clo1 = chunk_lo(c1)
        wait_chunk(c1, buf1, gsem1)
        cnt = chunk_tokens(buf1, clo1, clo1 + CHW, cnt)
        start_chunk(c1 + 2, buf1, gsem1)

        return cnt

    cnt = lax.fori_loop(0, (NCH + 3) // 2, chunk_iter, 0)

    def drain(i, _):
        pltpu.make_async_copy(rows_v.at[pl.ds(0, D)],
                              out_hbm.at[pl.ds(0, D)], osem).wait()
        return _

    lax.fori_loop(0, jnp.minimum(cnt, RING), drain, 0)


def kernel(token_index, e_weights, ln_gamma, ln_beta, weights_scale,
           norm_scale):
    gb = jnp.concatenate([ln_gamma, ln_beta])
    sc = jnp.concatenate([
        jnp.full((L,), weights_scale, jnp.float32),
        jnp.full((L,), norm_scale, jnp.float32),
    ])
    tl = jnp.pad(e_weights[LEFT_LO:, :], ((0, 448), (0, 0))).T
    flat = _embed_ln_kernel(token_index, e_weights.T, gb, sc, tl)
    return flat.reshape(B, D)


# speedup vs baseline: 1.0065x; 1.0065x over previous
import functools

import jax
import jax.numpy as jnp
from jax import lax
from jax.experimental import pallas as pl
from jax.experimental.pallas import tpu as pltpu
from jax.experimental.pallas import tpu_sc as plsc

V = 1000000
D = 64
B = 16384
L = 16
NC, NS = 2, 16
NW = NC * NS

CHW = 512
NCH = 61
SPAN = NCH * CHW
LEFT_LO = 999424
XT_LO = 999936

WCAP = 832
WPAD = 912
RING = 128

_MAGIC = 0x5F3759DF
_EPS = 1e-5


@functools.partial(
    pl.kernel,
    mesh=plsc.VectorSubcoreMesh(core_axis_name="c", subcore_axis_name="s"),
    out_type=jax.ShapeDtypeStruct((B * D,), jnp.float32),
    scratch_types=[
        pltpu.VMEM((B,), jnp.int32),
        pltpu.VMEM((WPAD,), jnp.int32),
        pltpu.VMEM((WPAD,), jnp.int32),
        pltpu.VMEM((D, CHW), jnp.float32),
        pltpu.VMEM((D, CHW), jnp.float32),
        pltpu.VMEM((RING * D,), jnp.float32),
        pltpu.VMEM((2 * D,), jnp.float32),
        pltpu.VMEM((2 * L,), jnp.float32),
        pltpu.SemaphoreType.DMA,
        pltpu.SemaphoreType.DMA,
        pltpu.SemaphoreType.DMA,
    ],
)
def _embed_ln_kernel(tok_hbm, tt_hbm, gb_hbm, sc_hbm, tl_hbm, out_hbm,
                     ids_v, wid_v, wpos_v, buf0, buf1,
                     rows_v, gb_v, sc_v, gsem0, gsem1, osem):
    w = lax.axis_index("s") * NC + lax.axis_index("c")
    lo = w * SPAN

    pltpu.sync_copy(tok_hbm, ids_v)
    pltpu.sync_copy(gb_hbm, gb_v)
    pltpu.sync_copy(sc_hbm, sc_v)

    lane = lax.iota(jnp.int32, L)

    dnums = lax.GatherDimensionNumbers(
        offset_dims=(), collapsed_slice_dims=(0,), start_index_map=(0,))

    def permute(x, p):
        return lax.gather(x, p[:, None], dnums, slice_sizes=(1,),
                          mode=lax.GatherScatterMode.PROMISE_IN_BOUNDS)

    xorperms = [lane ^ m for m in (8, 4, 2, 1)]

    def hsum(x):
        for p in xorperms:
            x = x + permute(x, p)
        return x

    shperms = [jnp.maximum(lane - sh, 0) for sh in (1, 2, 4, 8)]
    shsels = [jnp.where(lane >= sh, 1, 0) for sh in (1, 2, 4, 8)]

    def incl_prefix(x):
        for p, s in zip(shperms, shsels):
            x = x + jnp.where(s == 1, permute(x, p), 0)
        return x

    def rsqrt_vec(v):
        i = lax.bitcast_convert_type(v, jnp.int32)
        y = lax.bitcast_convert_type(jnp.int32(_MAGIC) - (i >> 1),
                                     jnp.float32)
        hv = v * 0.5
        for _ in range(3):
            y = y * (1.5 - hv * y * y)
        return y

    is_last = w == NW - 1
    elo = jnp.where(is_last, LEFT_LO, V)

    def scan1(b, cur):
        ids = ids_v[pl.ds(b * L, L)]
        poss = lane + b * L
        m = ((ids >= lo) & (ids < lo + SPAN)) | (ids >= elo)
        m01 = jnp.where(m, 1, 0)
        tot = hsum(m01)[0]

        def take(i, cur2):
            pref = incl_prefix(m01)
            sel = jnp.where((m01 == 1) & (pref == i + 1), 1, 0)
            tid = hsum(sel * ids)[0]
            tpos = hsum(sel * poss)[0]
            dst = jnp.minimum(cur2, WCAP - 1)
            wid_v[pl.ds(dst, L)] = jnp.full((L,), tid, jnp.int32)
            wpos_v[pl.ds(dst, L)] = jnp.full((L,), tpos, jnp.int32)
            return cur2 + 1

        return lax.fori_loop(0, tot, take, cur)

    def scan(b2, cur):
        for u in range(4):
            cur = scan1(b2 * 4 + u, cur)
        return cur

    n = lax.fori_loop(0, B // L // 4, scan, 0)
    n = jnp.minimum(n, WCAP)
    nv = (n + L - 1) // L

    gvecs = [gb_v[pl.ds(L * k, L)] for k in range(4)]
    bvecs = [gb_v[pl.ds(D + L * k, L)] for k in range(4)]
    wsv = sc_v[pl.ds(0, L)]
    nsv = sc_v[pl.ds(L, L)]
    lmasks = [jnp.where(lane == j, 1, 0) for j in range(L)]

    def ln_emit(x, tpos, cnt):
        s = (x[0] + x[1]) + (x[2] + x[3])
        q = (x[0] * x[0] + x[1] * x[1]) + (x[2] * x[2] + x[3] * x[3])
        mv = hsum(s) * (1.0 / D)
        ex2 = hsum(q) * (1.0 / D)
        var = ex2 - mv * mv
        rinv = rsqrt_vec(var + _EPS)

        @pl.when(cnt >= RING)
        def _():
            pltpu.make_async_copy(rows_v.at[pl.ds(0, D)],
                                  out_hbm.at[pl.ds(0, D)], osem).wait()

        slot = cnt & (RING - 1)
        for k in range(4):
            xh = (x[k] - mv) * rinv
            rows_v[pl.ds(slot * D + L * k, L)] = (
                x[k] * wsv + (xh * gvecs[k] + bvecs[k]) * nsv)
        pltpu.async_copy(rows_v.at[pl.ds(slot * D, D)],
                         out_hbm.at[pl.ds(tpos * D, D)], osem)

    def emit_token(buf, col, tpos, cnt):
        cw = (col >> 4) * L
        cl = jnp.full((L,), col & (L - 1), jnp.int32)
        x = []
        for k in range(4):
            acc = jnp.zeros((L,), jnp.float32)
            for j in range(L):
                spl = permute(buf[L * k + j, pl.ds(cw, L)], cl)
                acc = jnp.where(lmasks[j] == 1, spl, acc)
            x.append(acc)
        ln_emit(x, tpos, cnt)

    def chunk_tokens(buf, clo, chi, cnt):
        def vreg(vi, cnt2):
            ids = wid_v[pl.ds(vi * L, L)]
            inw = (lane + vi * L) < n
            m01 = jnp.where((ids >= clo) & (ids < chi) & inw, 1, 0)
            tot = hsum(m01)[0]

            def take(i, cnt3):
                pref = incl_prefix(m01)
                poss = wpos_v[pl.ds(vi * L, L)]
                sel = jnp.where((m01 == 1) & (pref == i + 1), 1, 0)
                tid = hsum(sel * ids)[0]
                tpos = hsum(sel * poss)[0]
                emit_token(buf, tid - clo, tpos, cnt3)
                return cnt3 + 1

            return lax.fori_loop(0, tot, take, cnt2)

        def vreg2(v2, cnt2):
            cnt2 = vreg(2 * v2, cnt2)
            return vreg(2 * v2 + 1, cnt2)

        return lax.fori_loop(0, (nv + 1) // 2, vreg2, cnt)

    nch_w = NCH + jnp.where(is_last, 2, 0)

    def chunk_lo(c):
        return jnp.where(c < NCH, lo + c * CHW, LEFT_LO + (c - NCH) * CHW)

    def start_chunk(c, buf, sem):
        @pl.when(c < NCH)
        def _main():
            off = pl.multiple_of(lo + c * CHW, 128)
            pltpu.async_copy(tt_hbm.at[:, pl.ds(off, CHW)], buf, sem)

        @pl.when((c >= NCH) & (c < nch_w))
        def _left():
            off = pl.multiple_of((c - NCH) * CHW, 128)
            pltpu.async_copy(tl_hbm.at[:, pl.ds(off, CHW)], buf, sem)

    def wait_chunk(c, buf, sem):
        @pl.when(c < nch_w)
        def _():
            pltpu.make_async_copy(tt_hbm.at[:, pl.ds(0, CHW)],
                                  buf, sem).wait()

    start_chunk(0, buf0, gsem0)
    start_chunk(1, buf1, gsem1)

    def chunk_iter(k, cnt):
        c0 = 2 * k
        clo0 = chunk_lo(c0)
        wait_chunk(c0, buf0, gsem0)
        cnt = chunk_tokens(buf0, clo0, clo0 + CHW, cnt)
        start_chunk(c0 + 2, buf0, gsem0)

        c1 = c0 + 1
        clo1 = chunk_lo(c1)
        wait_chunk(c1, buf1, gsem1)
        cnt = chunk_tokens(buf1, clo1, clo1 + CHW, cnt)
        start_chunk(c1 + 2, buf1, gsem1)

        return cnt

    cnt = lax.fori_loop(0, (NCH + 3) // 2, chunk_iter, 0)

    def drain(i, _):
        pltpu.make_async_copy(rows_v.at[pl.ds(0, D)],
                              out_hbm.at[pl.ds(0, D)], osem).wait()
        return _

    lax.fori_loop(0, jnp.minimum(cnt, RING), drain, 0)


def kernel(token_index, e_weights, ln_gamma, ln_beta, weights_scale,
           norm_scale):
    gb = jnp.concatenate([ln_gamma, ln_beta])
    sc = jnp.concatenate([
        jnp.full((L,), weights_scale, jnp.float32),
        jnp.full((L,), norm_scale, jnp.float32),
    ])
    tl = jnp.pad(e_weights[LEFT_LO:, :], ((0, 448), (0, 0))).T
    flat = _embed_ln_kernel(token_index, e_weights.T, gb, sc, tl)
    return flat.reshape(B, D)


# speedup vs baseline: 1.0258x; 1.0192x over previous
import functools

import jax
import jax.numpy as jnp
from jax import lax
from jax.experimental import pallas as pl
from jax.experimental.pallas import tpu as pltpu
from jax.experimental.pallas import tpu_sc as plsc

V = 1000000
D = 64
B = 16384
L = 16
NC, NS = 2, 16
NW = NC * NS

CHW = 512
NCH = 61
SPAN = NCH * CHW
LEFT_LO = 999424
XT_LO = 999936

WCAP = 832
WPAD = 912
RING = 128

_MAGIC = 0x5F3759DF
_EPS = 1e-5


@functools.partial(
    pl.kernel,
    mesh=plsc.VectorSubcoreMesh(core_axis_name="c", subcore_axis_name="s"),
    out_type=jax.ShapeDtypeStruct((B * D,), jnp.float32),
    scratch_types=[
        pltpu.VMEM((B,), jnp.int32),
        pltpu.VMEM((WPAD,), jnp.int32),
        pltpu.VMEM((WPAD,), jnp.int32),
        pltpu.VMEM((D, CHW), jnp.float32),
        pltpu.VMEM((D, CHW), jnp.float32),
        pltpu.VMEM((D, CHW), jnp.float32),
        pltpu.VMEM((RING * D,), jnp.float32),
        pltpu.VMEM((2 * D,), jnp.float32),
        pltpu.VMEM((2 * L,), jnp.float32),
        pltpu.SemaphoreType.DMA,
        pltpu.SemaphoreType.DMA,
        pltpu.SemaphoreType.DMA,
        pltpu.SemaphoreType.DMA,
    ],
)
def _embed_ln_kernel(tok_hbm, tt_hbm, gb_hbm, sc_hbm, tl_hbm, out_hbm,
                     ids_v, wid_v, wpos_v, buf0, buf1, buf2,
                     rows_v, gb_v, sc_v, gsem0, gsem1, gsem2, osem):
    w = lax.axis_index("s") * NC + lax.axis_index("c")
    lo = w * SPAN

    pltpu.sync_copy(tok_hbm, ids_v)
    pltpu.sync_copy(gb_hbm, gb_v)
    pltpu.sync_copy(sc_hbm, sc_v)

    lane = lax.iota(jnp.int32, L)

    dnums = lax.GatherDimensionNumbers(
        offset_dims=(), collapsed_slice_dims=(0,), start_index_map=(0,))

    def permute(x, p):
        return lax.gather(x, p[:, None], dnums, slice_sizes=(1,),
                          mode=lax.GatherScatterMode.PROMISE_IN_BOUNDS)

    xorperms = [lane ^ m for m in (8, 4, 2, 1)]

    def hsum(x):
        for p in xorperms:
            x = x + permute(x, p)
        return x

    shperms = [jnp.maximum(lane - sh, 0) for sh in (1, 2, 4, 8)]
    shsels = [jnp.where(lane >= sh, 1, 0) for sh in (1, 2, 4, 8)]

    def incl_prefix(x):
        for p, s in zip(shperms, shsels):
            x = x + jnp.where(s == 1, permute(x, p), 0)
        return x

    def rsqrt_vec(v):
        i = lax.bitcast_convert_type(v, jnp.int32)
        y = lax.bitcast_convert_type(jnp.int32(_MAGIC) - (i >> 1),
                                     jnp.float32)
        hv = v * 0.5
        for _ in range(3):
            y = y * (1.5 - hv * y * y)
        return y

    is_last = w == NW - 1
    elo = jnp.where(is_last, LEFT_LO, V)

    def scan1(b, cur):
        ids = ids_v[pl.ds(b * L, L)]
        poss = lane + b * L
        m = ((ids >= lo) & (ids < lo + SPAN)) | (ids >= elo)
        m01 = jnp.where(m, 1, 0)
        tot = hsum(m01)[0]

        def take(i, cur2):
            pref = incl_prefix(m01)
            sel = jnp.where((m01 == 1) & (pref == i + 1), 1, 0)
            tid = hsum(sel * ids)[0]
            tpos = hsum(sel * poss)[0]
            dst = jnp.minimum(cur2, WCAP - 1)
            wid_v[pl.ds(dst, L)] = jnp.full((L,), tid, jnp.int32)
            wpos_v[pl.ds(dst, L)] = jnp.full((L,), tpos, jnp.int32)
            return cur2 + 1

        return lax.fori_loop(0, tot, take, cur)

    def scan(b2, cur):
        for u in range(4):
            cur = scan1(b2 * 4 + u, cur)
        return cur

    n = lax.fori_loop(0, B // L // 4, scan, 0)
    n = jnp.minimum(n, WCAP)
    nv = (n + L - 1) // L

    gvecs = [gb_v[pl.ds(L * k, L)] for k in range(4)]
    bvecs = [gb_v[pl.ds(D + L * k, L)] for k in range(4)]
    wsv = sc_v[pl.ds(0, L)]
    nsv = sc_v[pl.ds(L, L)]
    lmasks = [jnp.where(lane == j, 1, 0) for j in range(L)]

    def ln_emit(x, tpos, cnt):
        s = (x[0] + x[1]) + (x[2] + x[3])
        q = (x[0] * x[0] + x[1] * x[1]) + (x[2] * x[2] + x[3] * x[3])
        mv = hsum(s) * (1.0 / D)
        ex2 = hsum(q) * (1.0 / D)
        var = ex2 - mv * mv
        rinv = rsqrt_vec(var + _EPS)

        @pl.when(cnt >= RING)
        def _():
            pltpu.make_async_copy(rows_v.at[pl.ds(0, D)],
                                  out_hbm.at[pl.ds(0, D)], osem).wait()

        slot = cnt & (RING - 1)
        for k in range(4):
            xh = (x[k] - mv) * rinv
            rows_v[pl.ds(slot * D + L * k, L)] = (
                x[k] * wsv + (xh * gvecs[k] + bvecs[k]) * nsv)
        pltpu.async_copy(rows_v.at[pl.ds(slot * D, D)],
                         out_hbm.at[pl.ds(tpos * D, D)], osem)

    def emit_token(buf, col, tpos, cnt):
        cw = (col >> 4) * L
        cl = jnp.full((L,), col & (L - 1), jnp.int32)
        x = []
        for k in range(4):
            acc = jnp.zeros((L,), jnp.float32)
            for j in range(L):
                spl = permute(buf[L * k + j, pl.ds(cw, L)], cl)
                acc = jnp.where(lmasks[j] == 1, spl, acc)
            x.append(acc)
        ln_emit(x, tpos, cnt)

    def chunk_tokens(buf, clo, chi, cnt):
        def vreg(vi, cnt2):
            ids = wid_v[pl.ds(vi * L, L)]
            inw = (lane + vi * L) < n
            m01 = jnp.where((ids >= clo) & (ids < chi) & inw, 1, 0)
            tot = hsum(m01)[0]

            def take(i, cnt3):
                pref = incl_prefix(m01)
                poss = wpos_v[pl.ds(vi * L, L)]
                sel = jnp.where((m01 == 1) & (pref == i + 1), 1, 0)
                tid = hsum(sel * ids)[0]
                tpos = hsum(sel * poss)[0]
                emit_token(buf, tid - clo, tpos, cnt3)
                return cnt3 + 1

            return lax.fori_loop(0, tot, take, cnt2)

        def vreg2(v2, cnt2):
            cnt2 = vreg(2 * v2, cnt2)
            return vreg(2 * v2 + 1, cnt2)

        return lax.fori_loop(0, (nv + 1) // 2, vreg2, cnt)

    nm_w = NCH + jnp.where(is_last, 1, 0)
    nch_w = NCH + jnp.where(is_last, 2, 0)

    def chunk_lo(c):
        return jnp.where(c < nm_w, lo + c * CHW, XT_LO)

    def start_chunk(c, buf, sem):
        @pl.when(c < nm_w)
        def _main():
            off = pl.multiple_of(lo + c * CHW, 128)
            pltpu.async_copy(tt_hbm.at[:, pl.ds(off, CHW)], buf, sem)

        @pl.when((c >= nm_w) & (c < nch_w))
        def _left():
            pltpu.async_copy(tl_hbm.at[:, pl.ds(0, CHW)], buf, sem)

    def wait_chunk(c, buf, sem):
        @pl.when(c < nch_w)
        def _():
            pltpu.make_async_copy(tt_hbm.at[:, pl.ds(0, CHW)],
                                  buf, sem).wait()

    start_chunk(0, buf0, gsem0)
    start_chunk(1, buf1, gsem1)
    start_chunk(2, buf2, gsem2)

    def chunk_iter(k, cnt):
        c0 = 3 * k
        clo0 = chunk_lo(c0)
        wait_chunk(c0, buf0, gsem0)
        cnt = chunk_tokens(buf0, clo0, clo0 + CHW, cnt)
        start_chunk(c0 + 3, buf0, gsem0)

        c1 = c0 + 1
        clo1 = chunk_lo(c1)
        wait_chunk(c1, buf1, gsem1)
        cnt = chunk_tokens(buf1, clo1, clo1 + CHW, cnt)
        start_chunk(c1 + 3, buf1, gsem1)

        c2 = c0 + 2
        clo2 = chunk_lo(c2)
        wait_chunk(c2, buf2, gsem2)
        cnt = chunk_tokens(buf2, clo2, clo2 + CHW, cnt)
        start_chunk(c2 + 3, buf2, gsem2)

        return cnt

    cnt = lax.fori_loop(0, (NCH + 2) // 3, chunk_iter, 0)

    def drain(i, _):
        pltpu.make_async_copy(rows_v.at[pl.ds(0, D)],
                              out_hbm.at[pl.ds(0, D)], osem).wait()
        return _

    lax.fori_loop(0, jnp.minimum(cnt, RING), drain, 0)


def kernel(token_index, e_weights, ln_gamma, ln_beta, weights_scale,
           norm_scale):
    gb = jnp.concatenate([ln_gamma, ln_beta])
    sc = jnp.concatenate([
        jnp.full((L,), weights_scale, jnp.float32),
        jnp.full((L,), norm_scale, jnp.float32),
    ])
    tl = jnp.pad(e_weights[XT_LO:, :], ((0, CHW - D), (0, 0))).T
    flat = _embed_ln_kernel(token_index, e_weights.T, gb, sc, tl)
    return flat.reshape(B, D)


# speedup vs baseline: 1.0516x; 1.0251x over previous
import functools

import jax
import jax.numpy as jnp
from jax import lax
from jax.experimental import pallas as pl
from jax.experimental.pallas import tpu as pltpu
from jax.experimental.pallas import tpu_sc as plsc

V = 1000000
D = 64
B = 16384
L = 16
NC, NS = 2, 16
NW = NC * NS

CHW = 512
NCH = 61
SPAN = NCH * CHW
LEFT_LO = 999424
XT_LO = 999936

WCAP = 832
WPAD = 912
RING = 128

_MAGIC = 0x5F3759DF
_EPS = 1e-5


@functools.partial(
    pl.kernel,
    mesh=plsc.VectorSubcoreMesh(core_axis_name="c", subcore_axis_name="s"),
    out_type=jax.ShapeDtypeStruct((B * D,), jnp.float32),
    scratch_types=[
        pltpu.VMEM((B,), jnp.int32),
        pltpu.VMEM((WPAD,), jnp.int32),
        pltpu.VMEM((WPAD,), jnp.int32),
        pltpu.VMEM((D, CHW), jnp.float32),
        pltpu.VMEM((D, CHW), jnp.float32),
        pltpu.VMEM((D, CHW), jnp.float32),
        pltpu.VMEM((RING * D,), jnp.float32),
        pltpu.VMEM((2 * D,), jnp.float32),
        pltpu.VMEM((2 * L,), jnp.float32),
        pltpu.SemaphoreType.DMA,
        pltpu.SemaphoreType.DMA,
        pltpu.SemaphoreType.DMA,
        pltpu.SemaphoreType.DMA,
    ],
)
def _embed_ln_kernel(tok_hbm, tt_hbm, gb_hbm, sc_hbm, tl_hbm, out_hbm,
                     ids_v, wid_v, wpos_v, buf0, buf1, buf2,
                     rows_v, gb_v, sc_v, gsem0, gsem1, gsem2, osem):
    w = lax.axis_index("s") * NC + lax.axis_index("c")
    lo = w * SPAN

    pltpu.sync_copy(tok_hbm, ids_v)
    pltpu.sync_copy(gb_hbm, gb_v)
    pltpu.sync_copy(sc_hbm, sc_v)

    lane = lax.iota(jnp.int32, L)

    dnums = lax.GatherDimensionNumbers(
        offset_dims=(), collapsed_slice_dims=(0,), start_index_map=(0,))

    def permute(x, p):
        return lax.gather(x, p[:, None], dnums, slice_sizes=(1,),
                          mode=lax.GatherScatterMode.PROMISE_IN_BOUNDS)

    xorperms = [lane ^ m for m in (8, 4, 2, 1)]

    def hsum(x):
        for p in xorperms:
            x = x + permute(x, p)
        return x

    shperms = [jnp.maximum(lane - sh, 0) for sh in (1, 2, 4, 8)]
    shsels = [jnp.where(lane >= sh, 1, 0) for sh in (1, 2, 4, 8)]

    def incl_prefix(x):
        for p, s in zip(shperms, shsels):
            x = x + jnp.where(s == 1, permute(x, p), 0)
        return x

    def rsqrt_vec(v):
        i = lax.bitcast_convert_type(v, jnp.int32)
        y = lax.bitcast_convert_type(jnp.int32(_MAGIC) - (i >> 1),
                                     jnp.float32)
        hv = v * 0.5
        for _ in range(3):
            y = y * (1.5 - hv * y * y)
        return y

    is_last = w == NW - 1
    elo = jnp.where(is_last, LEFT_LO, V)

    nm_w = NCH + jnp.where(is_last, 1, 0)
    nch_w = NCH + jnp.where(is_last, 2, 0)

    def chunk_lo(c):
        return jnp.where(c < nm_w, lo + c * CHW, XT_LO)

    def start_chunk(c, buf, sem):
        @pl.when(c < nm_w)
        def _main():
            off = pl.multiple_of(lo + c * CHW, 128)
            pltpu.async_copy(tt_hbm.at[:, pl.ds(off, CHW)], buf, sem)

        @pl.when((c >= nm_w) & (c < nch_w))
        def _left():
            pltpu.async_copy(tl_hbm.at[:, pl.ds(0, CHW)], buf, sem)

    def wait_chunk(c, buf, sem):
        @pl.when(c < nch_w)
        def _():
            pltpu.make_async_copy(tt_hbm.at[:, pl.ds(0, CHW)],
                                  buf, sem).wait()

    start_chunk(0, buf0, gsem0)
    start_chunk(1, buf1, gsem1)
    start_chunk(2, buf2, gsem2)

    def scan1(b, cur):
        ids = ids_v[pl.ds(b * L, L)]
        poss = lane + b * L
        m = ((ids >= lo) & (ids < lo + SPAN)) | (ids >= elo)
        m01 = jnp.where(m, 1, 0)
        tot = hsum(m01)[0]

        def take(i, cur2):
            pref = incl_prefix(m01)
            sel = jnp.where((m01 == 1) & (pref == i + 1), 1, 0)
            tid = hsum(sel * ids)[0]
            tpos = hsum(sel * poss)[0]
            dst = jnp.minimum(cur2, WCAP - 1)
            wid_v[pl.ds(dst, L)] = jnp.full((L,), tid, jnp.int32)
            wpos_v[pl.ds(dst, L)] = jnp.full((L,), tpos, jnp.int32)
            return cur2 + 1

        return lax.fori_loop(0, tot, take, cur)

    def scan(b2, cur):
        for u in range(4):
            cur = scan1(b2 * 4 + u, cur)
        return cur

    n = lax.fori_loop(0, B // L // 4, scan, 0)
    n = jnp.minimum(n, WCAP)
    nv = (n + L - 1) // L

    gvecs = [gb_v[pl.ds(L * k, L)] for k in range(4)]
    bvecs = [gb_v[pl.ds(D + L * k, L)] for k in range(4)]
    wsv = sc_v[pl.ds(0, L)]
    nsv = sc_v[pl.ds(L, L)]
    lmasks = [jnp.where(lane == j, 1, 0) for j in range(L)]

    def ln_emit(x, tpos, cnt):
        s = (x[0] + x[1]) + (x[2] + x[3])
        q = (x[0] * x[0] + x[1] * x[1]) + (x[2] * x[2] + x[3] * x[3])
        mv = hsum(s) * (1.0 / D)
        ex2 = hsum(q) * (1.0 / D)
        var = ex2 - mv * mv
        rinv = rsqrt_vec(var + _EPS)

        @pl.when(cnt >= RING)
        def _():
            pltpu.make_async_copy(rows_v.at[pl.ds(0, D)],
                                  out_hbm.at[pl.ds(0, D)], osem).wait()

        slot = cnt & (RING - 1)
        for k in range(4):
            xh = (x[k] - mv) * rinv
            rows_v[pl.ds(slot * D + L * k, L)] = (
                x[k] * wsv + (xh * gvecs[k] + bvecs[k]) * nsv)
        pltpu.async_copy(rows_v.at[pl.ds(slot * D, D)],
                         out_hbm.at[pl.ds(tpos * D, D)], osem)

    def emit_token(buf, col, tpos, cnt):
        cw = (col >> 4) * L
        cl = jnp.full((L,), col & (L - 1), jnp.int32)
        x = []
        for k in range(4):
            acc = jnp.zeros((L,), jnp.float32)
            for j in range(L):
                spl = permute(buf[L * k + j, pl.ds(cw, L)], cl)
                acc = jnp.where(lmasks[j] == 1, spl, acc)
            x.append(acc)
        ln_emit(x, tpos, cnt)

    def chunk_tokens(buf, clo, chi, cnt):
        def vreg(vi, cnt2):
            ids = wid_v[pl.ds(vi * L, L)]
            inw = (lane + vi * L) < n
            m01 = jnp.where((ids >= clo) & (ids < chi) & inw, 1, 0)
            tot = hsum(m01)[0]

            def take(i, cnt3):
                pref = incl_prefix(m01)
                poss = wpos_v[pl.ds(vi * L, L)]
                sel = jnp.where((m01 == 1) & (pref == i + 1), 1, 0)
                tid = hsum(sel * ids)[0]
                tpos = hsum(sel * poss)[0]
                emit_token(buf, tid - clo, tpos, cnt3)
                return cnt3 + 1

            return lax.fori_loop(0, tot, take, cnt2)

        def vreg2(v2, cnt2):
            cnt2 = vreg(2 * v2, cnt2)
            return vreg(2 * v2 + 1, cnt2)

        return lax.fori_loop(0, (nv + 1) // 2, vreg2, cnt)

    def chunk_iter(k, cnt):
        c0 = 3 * k
        clo0 = chunk_lo(c0)
        wait_chunk(c0, buf0, gsem0)
        cnt = chunk_tokens(buf0, clo0, clo0 + CHW, cnt)
        start_chunk(c0 + 3, buf0, gsem0)

        c1 = c0 + 1
        clo1 = chunk_lo(c1)
        wait_chunk(c1, buf1, gsem1)
        cnt = chunk_tokens(buf1, clo1, clo1 + CHW, cnt)
        start_chunk(c1 + 3, buf1, gsem1)

        c2 = c0 + 2
        clo2 = chunk_lo(c2)
        wait_chunk(c2, buf2, gsem2)
        cnt = chunk_tokens(buf2, clo2, clo2 + CHW, cnt)
        start_chunk(c2 + 3, buf2, gsem2)

        return cnt

    cnt = lax.fori_loop(0, (NCH + 2) // 3, chunk_iter, 0)

    def drain(i, _):
        pltpu.make_async_copy(rows_v.at[pl.ds(0, D)],
                              out_hbm.at[pl.ds(0, D)], osem).wait()
        return _

    lax.fori_loop(0, jnp.minimum(cnt, RING), drain, 0)


def kernel(token_index, e_weights, ln_gamma, ln_beta, weights_scale,
           norm_scale):
    gb = jnp.concatenate([ln_gamma, ln_beta])
    sc = jnp.concatenate([
        jnp.full((L,), weights_scale, jnp.float32),
        jnp.full((L,), norm_scale, jnp.float32),
    ])
    tl = jnp.pad(e_weights[XT_LO:, :], ((0, CHW - D), (0, 0))).T
    flat = _embed_ln_kernel(token_index, e_weights.T, gb, sc, tl)
    return flat.reshape(B, D)
